# baseline (speedup 1.0000x reference)
import functools

import jax
import jax.numpy as jnp
from jax import lax
from jax.experimental import pallas as pl
from jax.experimental.pallas import tpu as pltpu
from jax.experimental.pallas import tpu_sc as plsc

_PROMPT_LAYER_INDICES = (0,)
_NC, _NS, _L = 2, 16, 16
_NW = _NC * _NS


@functools.cache
def _build_sc_gather(B, E, NT, P, zero_prefix):
    PR = B * P
    assert PR % _NW == 0
    pr_w = PR // _NW
    assert pr_w == _L

    mesh = plsc.VectorSubcoreMesh(core_axis_name="c", subcore_axis_name="s")

    def body(tab_hbm, pidx_hbm, q_hbm, idx_v, pbuf, sg, sp):
        wid = lax.axis_index("s") * _NC + lax.axis_index("c")
        if zero_prefix:
            def zcol(j, carry):
                for r in range(_L):
                    pbuf[r, pl.ds(j * _L, _L)] = jnp.zeros((_L,), jnp.float32)
                return carry
            lax.fori_loop(0, E // _L, zcol, 0)
        else:
            pltpu.sync_copy(pidx_hbm.at[pl.ds(wid * pr_w, pr_w)], idx_v)
            gather = pltpu.make_async_copy(tab_hbm.at[idx_v], pbuf, sg)
            gather.start()
            gather.wait()
        store = pltpu.make_async_copy(
            pbuf, q_hbm.at[pl.ds(wid * pr_w, pr_w)], sp)
        store.start()
        store.wait()

    return pl.kernel(
        body,
        out_type=jax.ShapeDtypeStruct((PR, E), jnp.float32),
        mesh=mesh,
        scratch_types=[
            pltpu.VMEM((_L,), jnp.int32),
            pltpu.VMEM((pr_w, E), jnp.float32),
            pltpu.SemaphoreType.DMA,
            pltpu.SemaphoreType.DMA,
        ],
    )


@functools.cache
def _build_tc_assemble(B, T, E, P):
    BR = P
    assert T % BR == 0
    jn = T // BR + 1

    def body(q_ref, in_ref, o_ref):
        j = pl.program_id(1)

        @pl.when(j == 0)
        def _prefix():
            o_ref[...] = q_ref[...]

        @pl.when(j != 0)
        def _bulk():
            o_ref[...] = in_ref[...]

    return pl.pallas_call(
        body,
        grid=(B, jn),
        in_specs=[
            pl.BlockSpec((BR, E), lambda b, j: (b, 0)),
            pl.BlockSpec((BR, E), lambda b, j: (b * (jn - 1)
                                                + jnp.maximum(j - 1, 0), 0)),
        ],
        out_specs=pl.BlockSpec((BR, E), lambda b, j: (b * jn + j, 0)),
        out_shape=jax.ShapeDtypeStruct((B * (P + T), E), jnp.float32),
    )


def kernel(input_embedding, layer_idx, task_ids, prompt_table):
    B, T, E = input_embedding.shape
    NT, P, _ = prompt_table.shape
    if P == 0:
        return input_embedding
    active = jnp.any(
        jnp.asarray(_PROMPT_LAYER_INDICES, jnp.int32)
        == jnp.asarray(layer_idx, jnp.int32))
    in_rows = input_embedding.reshape(B * T, E)
    tab_rows = prompt_table.reshape(NT * P, E)
    prow_idx = (task_ids.astype(jnp.int32)[:, None] * P
                + jnp.arange(P, dtype=jnp.int32)[None, :]).reshape(B * P)
    q = lax.cond(
        active,
        lambda a, b: _build_sc_gather(B, E, NT, P, False)(a, b),
        lambda a, b: _build_sc_gather(B, E, NT, P, True)(a, b),
        tab_rows, prow_idx)
    out = _build_tc_assemble(B, T, E, P)(q, in_rows)
    return out.reshape(B, P + T, E)
